# Initial kernel scaffold; baseline (speedup 1.0000x reference)
import jax
import jax.numpy as jnp
from jax.experimental import pallas as pl


def kernel(x, c1_Wn, c1_Wr, c1_br, s1_g, s1_b, s1_W, s1_bias, c2_W1, c2_b1, c2_g, c2_beta, c2_W2, c2_b2, c3_W1, c3_b1, c3_g, c3_beta, c3_W2, c3_b2, l1_W, l1_b, l2_W, l2_b, o_W, o_b, edge_index, batch):
    raise NotImplementedError("write your pallas kernel here")


# speedup vs baseline: 4.5014x; 4.5014x over previous
import functools

import jax
import jax.numpy as jnp
from jax import lax
from jax.experimental import pallas as pl
from jax.experimental.pallas import tpu as pltpu
from jax.experimental.pallas import tpu_sc as plsc

N_NODES = 10000
N_EDGES = 320000
IN_CH = 128
C4 = 512
C8 = 1024
MAX_DEG = 10
NPAD = 10240

NC = 2
NS = 16
NT = NC * NS
EPT = N_EDGES // NT
EB = 200
NBATCH = EPT // EB
RPT = NPAD // NS

_f32 = jnp.float32
_sc_mesh = dict(core_axis_name="c", subcore_axis_name="s")


def _sc_deg_body(dst_hbm, zr_hbm, ones_hbm, deg_hbm, ones_v, dst_v, dacc, sem):
    core = lax.axis_index("c")
    sid = lax.axis_index("s")
    tid = core * NS + sid
    base = sid * RPT

    pltpu.sync_copy(ones_hbm, ones_v)
    pltpu.sync_copy(zr_hbm, dacc.at[pl.ds(base, RPT)])
    plsc.subcore_barrier()
    ebase = tid * EPT

    @pl.loop(0, NBATCH)
    def _(i):
        off = ebase + i * EB
        pltpu.sync_copy(dst_hbm.at[pl.ds(off, EB)], dst_v)
        pltpu.sync_copy(ones_v, dacc.at[dst_v], add=True)

    plsc.subcore_barrier()
    pltpu.sync_copy(dacc.at[pl.ds(base, RPT)],
                    deg_hbm.at[core].at[pl.ds(base, RPT)])


def _sc_deg(dst_r, zrows, ones_rows):
    k = pl.kernel(
        _sc_deg_body,
        out_type=jax.ShapeDtypeStruct((NC, NPAD, IN_CH), _f32),
        mesh=plsc.VectorSubcoreMesh(**_sc_mesh),
        scratch_types=[
            pltpu.VMEM((EB, IN_CH), _f32),
            pltpu.VMEM((EB,), jnp.int32),
            pltpu.VMEM_SHARED((NPAD, IN_CH), _f32),
            pltpu.SemaphoreType.DMA,
        ],
    )
    return k(dst_r, zrows, ones_rows)


def _sc_agg_x_body(x_hbm, src_hbm, dst_hbm, zr_hbm, agg_hbm,
                   src_v, dst_v, rows_v, acc, sem):
    core = lax.axis_index("c")
    sid = lax.axis_index("s")
    tid = core * NS + sid
    base = sid * RPT

    pltpu.sync_copy(zr_hbm, acc.at[pl.ds(base, RPT)])
    plsc.subcore_barrier()
    ebase = tid * EPT

    @pl.loop(0, NBATCH)
    def _(i):
        off = ebase + i * EB
        pltpu.sync_copy(src_hbm.at[pl.ds(off, EB)], src_v)
        pltpu.sync_copy(dst_hbm.at[pl.ds(off, EB)], dst_v)
        pltpu.async_copy(x_hbm.at[src_v], rows_v, sem).wait()
        pltpu.sync_copy(rows_v, acc.at[dst_v], add=True)

    plsc.subcore_barrier()
    pltpu.sync_copy(acc.at[pl.ds(base, RPT)],
                    agg_hbm.at[core].at[pl.ds(base, RPT)])


def _sc_agg_x(x, src_r, dst_r, zrows):
    k = pl.kernel(
        _sc_agg_x_body,
        out_type=jax.ShapeDtypeStruct((NC, NPAD, IN_CH), _f32),
        mesh=plsc.VectorSubcoreMesh(**_sc_mesh),
        scratch_types=[
            pltpu.VMEM((EB,), jnp.int32),
            pltpu.VMEM((EB,), jnp.int32),
            pltpu.VMEM((EB, IN_CH), _f32),
            pltpu.VMEM_SHARED((NPAD, IN_CH), _f32),
            pltpu.SemaphoreType.DMA,
        ],
    )
    return k(x, src_r, dst_r, zrows)


def _sc_agg_h_body(h0_hbm, h1_hbm, h2_hbm, h3_hbm, src_hbm, dst_hbm, zr_hbm,
                   agg_hbm, src_v, dst_v, rows_v, acc, sem):
    core = lax.axis_index("c")
    sid = lax.axis_index("s")
    tid = core * NS + sid
    base = sid * RPT
    ebase = tid * EPT

    for c, hc in enumerate((h0_hbm, h1_hbm, h2_hbm, h3_hbm)):
        pltpu.sync_copy(zr_hbm, acc.at[pl.ds(base, RPT)])
        plsc.subcore_barrier()

        @pl.loop(0, NBATCH)
        def _(i):
            off = ebase + i * EB
            pltpu.sync_copy(src_hbm.at[pl.ds(off, EB)], src_v)
            pltpu.sync_copy(dst_hbm.at[pl.ds(off, EB)], dst_v)
            pltpu.async_copy(hc.at[src_v], rows_v, sem).wait()
            pltpu.sync_copy(rows_v, acc.at[dst_v], add=True)

        plsc.subcore_barrier()
        pltpu.sync_copy(acc.at[pl.ds(base, RPT)],
                        agg_hbm.at[c].at[core].at[pl.ds(base, RPT)])


def _sc_agg_h(h0, h1, h2, h3, src_r, dst_r, zrows):
    k = pl.kernel(
        _sc_agg_h_body,
        out_type=jax.ShapeDtypeStruct((4, NC, NPAD, IN_CH), _f32),
        mesh=plsc.VectorSubcoreMesh(**_sc_mesh),
        scratch_types=[
            pltpu.VMEM((EB,), jnp.int32),
            pltpu.VMEM((EB,), jnp.int32),
            pltpu.VMEM((EB, IN_CH), _f32),
            pltpu.VMEM_SHARED((NPAD, IN_CH), _f32),
            pltpu.SemaphoreType.DMA,
        ],
    )
    return k(h0, h1, h2, h3, src_r, dst_r, zrows)


NB = 512


def _tc_a_body(x_ref, agg_ref, deg_ref, wflat_ref, br_ref, g_ref, b_ref,
               sW_ref, sb_ref, h0_ref, h1_ref, h2_ref, h3_ref):
    ha = agg_ref[0] + agg_ref[1]
    xb = x_ref[...]
    degf = deg_ref[0] + deg_ref[1]
    d = jnp.clip(degf[:, 0:1].astype(jnp.int32), 0, MAX_DEG)
    oh = (d == lax.broadcasted_iota(jnp.int32, (1, MAX_DEG + 1), 1))
    oh = oh.astype(_f32)
    hx = jnp.concatenate([ha, xb], axis=1)
    z = jnp.concatenate([oh[:, dd:dd + 1] * hx for dd in range(MAX_DEG + 1)],
                        axis=1)
    mf = jnp.dot(z, wflat_ref[...], preferred_element_type=_f32)
    mf = mf + jnp.dot(oh, br_ref[...], preferred_element_type=_f32)
    y = jnp.maximum(mf * g_ref[...] + b_ref[...], 0.0)
    h = jnp.dot(y, sW_ref[...], preferred_element_type=_f32) + sb_ref[...]
    h = jnp.maximum(h, 0.0)
    h0_ref[...] = h[:, 0:128]
    h1_ref[...] = h[:, 128:256]
    h2_ref[...] = h[:, 256:384]
    h3_ref[...] = h[:, 384:512]


def _tc_a(x_pad, aggx, dega, wflat, br, gscale, bshift, s1_W, s1_bias):
    grid = (NPAD // NB,)
    const = lambda i: (0, 0)
    out_sd = jax.ShapeDtypeStruct((NPAD, IN_CH), _f32)
    return pl.pallas_call(
        _tc_a_body,
        grid=grid,
        in_specs=[
            pl.BlockSpec((NB, IN_CH), lambda i: (i, 0)),
            pl.BlockSpec((NC, NB, IN_CH), lambda i: (0, i, 0)),
            pl.BlockSpec((NC, NB, IN_CH), lambda i: (0, i, 0)),
            pl.BlockSpec(((MAX_DEG + 1) * 256, C4), const),
            pl.BlockSpec((MAX_DEG + 1, C4), const),
            pl.BlockSpec((1, C4), const),
            pl.BlockSpec((1, C4), const),
            pl.BlockSpec((C4, C4), const),
            pl.BlockSpec((1, C4), const),
        ],
        out_specs=[pl.BlockSpec((NB, IN_CH), lambda i: (i, 0))] * 4,
        out_shape=[out_sd] * 4,
    )(x_pad, aggx, dega, wflat, br, gscale, bshift, s1_W, s1_bias)


def _tc_b_body(h0, h1, h2, h3, aggh_ref,
               c2W1, c2b1, c2g, c2be, c2W2, c2b2,
               c3W1, c3b1, c3g, c3be, c3W2, c3b2,
               l1a, l1b_, l1bias, l2W, l2b, oW, ob, out_ref):
    hin = jnp.concatenate(
        [h0[...] + aggh_ref[0, 0] + aggh_ref[0, 1],
         h1[...] + aggh_ref[1, 0] + aggh_ref[1, 1],
         h2[...] + aggh_ref[2, 0] + aggh_ref[2, 1],
         h3[...] + aggh_ref[3, 0] + aggh_ref[3, 1]], axis=1)

    def gin(W1, b1, g, be, W2, b2):
        t = jnp.dot(hin, W1[...], preferred_element_type=_f32) + b1[...]
        t = jnp.maximum(t * g[...] + be[...], 0.0)
        t = jnp.dot(t, W2[...], preferred_element_type=_f32) + b2[...]
        return jnp.maximum(t, 0.0)

    x1 = gin(c2W1, c2b1, c2g, c2be, c2W2, c2b2)
    x2 = gin(c3W1, c3b1, c3g, c3be, c3W2, c3b2)
    y = (jnp.dot(x1, l1a[...], preferred_element_type=_f32)
         + jnp.dot(x2, l1b_[...], preferred_element_type=_f32) + l1bias[...])
    y = jnp.dot(y, l2W[...], preferred_element_type=_f32) + l2b[...]
    y = jnp.dot(y, oW[...], preferred_element_type=_f32) + ob[...]
    out_ref[...] = jax.nn.sigmoid(y)


def _tc_b(h_chunks, aggh, weights):
    grid = (NPAD // NB,)
    const = lambda i: (0, 0)
    wspecs = []
    for w in weights:
        if w.ndim == 2:
            wspecs.append(pl.BlockSpec(w.shape, const))
        else:
            raise AssertionError
    return pl.pallas_call(
        _tc_b_body,
        grid=grid,
        in_specs=[pl.BlockSpec((NB, IN_CH), lambda i: (i, 0))] * 4
        + [pl.BlockSpec((4, NC, NB, IN_CH), lambda i: (0, 0, i, 0))]
        + wspecs,
        out_specs=pl.BlockSpec((NB, IN_CH), lambda i: (i, 0)),
        out_shape=jax.ShapeDtypeStruct((NPAD, IN_CH), _f32),
    )(*h_chunks, aggh, *weights)


def kernel(x, c1_Wn, c1_Wr, c1_br, s1_g, s1_b, s1_W, s1_bias,
           c2_W1, c2_b1, c2_g, c2_beta, c2_W2, c2_b2,
           c3_W1, c3_b1, c3_g, c3_beta, c3_W2, c3_b2,
           l1_W, l1_b, l2_W, l2_b, o_W, o_b, edge_index, batch):
    del batch
    inv = 1.0 / jnp.sqrt(jnp.float32(1.0 + 1e-5))
    src_r = edge_index[0]
    dst_r = edge_index[1]
    x_pad = jnp.pad(x, ((0, NPAD - N_NODES), (0, 0)))
    zrows = jnp.zeros((RPT, IN_CH), _f32)
    ones_rows = jnp.ones((EB, IN_CH), _f32)

    dega = _sc_deg(dst_r, zrows, ones_rows)
    aggx = _sc_agg_x(x_pad, src_r, dst_r, zrows)

    wflat = jnp.concatenate([c1_Wn, c1_Wr], axis=1).reshape(
        (MAX_DEG + 1) * 256, C4)
    h0, h1, h2, h3 = _tc_a(
        x_pad, aggx, dega, wflat, c1_br,
        (s1_g * inv).reshape(1, C4), s1_b.reshape(1, C4),
        s1_W, s1_bias.reshape(1, C4))

    aggh = _sc_agg_h(h0, h1, h2, h3, src_r, dst_r, zrows)

    weights = (
        c2_W1, c2_b1.reshape(1, C8), (c2_g * inv).reshape(1, C8),
        c2_beta.reshape(1, C8), c2_W2, c2_b2.reshape(1, C8),
        c3_W1, c3_b1.reshape(1, C8), (c3_g * inv).reshape(1, C8),
        c3_beta.reshape(1, C8), c3_W2, c3_b2.reshape(1, C8),
        l1_W[:C8], l1_W[C8:], l1_b.reshape(1, C8),
        l2_W, l2_b.reshape(1, C4), o_W, o_b.reshape(1, IN_CH),
    )
    out = _tc_b((h0, h1, h2, h3), aggh, weights)
    return out[:N_NODES]


# speedup vs baseline: 4.7825x; 1.0624x over previous
import functools

import jax
import jax.numpy as jnp
from jax import lax
from jax.experimental import pallas as pl
from jax.experimental.pallas import tpu as pltpu
from jax.experimental.pallas import tpu_sc as plsc

N_NODES = 10000
N_EDGES = 320000
IN_CH = 128
C4 = 512
C8 = 1024
MAX_DEG = 10
NPAD = 10240

NC = 2
NS = 16
NT = NC * NS
EPT = N_EDGES // NT
EB = 80
NBATCH = EPT // EB
RING = 4
NMAIN = (NBATCH // RING) * RING
EBD = 200
NBD = EPT // EBD
RPT = NPAD // NS

_f32 = jnp.float32
_sc_mesh = dict(core_axis_name="c", subcore_axis_name="s")


def _sc_deg_body(dst_hbm, zr_hbm, ones_hbm, deg_hbm, ones_v, dst_v, dacc, sem):
    core = lax.axis_index("c")
    sid = lax.axis_index("s")
    tid = core * NS + sid
    base = sid * RPT

    pltpu.sync_copy(ones_hbm, ones_v)
    pltpu.sync_copy(zr_hbm, dacc.at[pl.ds(base, RPT)])
    plsc.subcore_barrier()
    ebase = tid * EPT

    @pl.loop(0, NBD)
    def _(i):
        off = ebase + i * EBD
        pltpu.sync_copy(dst_hbm.at[pl.ds(off, EBD)], dst_v)
        pltpu.sync_copy(ones_v, dacc.at[dst_v], add=True)

    plsc.subcore_barrier()
    pltpu.sync_copy(dacc.at[pl.ds(base, RPT)],
                    deg_hbm.at[core].at[pl.ds(base, RPT)])


def _sc_deg(dst_r, zrows, ones_rows):
    k = pl.kernel(
        _sc_deg_body,
        out_type=jax.ShapeDtypeStruct((NC, NPAD, IN_CH), _f32),
        mesh=plsc.VectorSubcoreMesh(**_sc_mesh),
        scratch_types=[
            pltpu.VMEM((EBD, IN_CH), _f32),
            pltpu.VMEM((EBD,), jnp.int32),
            pltpu.VMEM_SHARED((NPAD, IN_CH), _f32),
            pltpu.SemaphoreType.DMA,
        ],
    )
    return k(dst_r, zrows, ones_rows)


def _ring_agg(table_hbm, src_hbm, dst_hbm, ebase, srcs, dsts, rows, sems,
              acc):
    def issue(b, i):
        off = ebase + i * EB
        pltpu.sync_copy(src_hbm.at[pl.ds(off, EB)], srcs[b])
        pltpu.sync_copy(dst_hbm.at[pl.ds(off, EB)], dsts[b])
        pltpu.async_copy(table_hbm.at[srcs[b]], rows[b], sems[b])

    def drain_scatter(b):
        pltpu.make_async_copy(table_hbm.at[srcs[b]], rows[b], sems[b]).wait()
        pltpu.sync_copy(rows[b], acc.at[dsts[b]], add=True)

    return issue, drain_scatter


def _sc_agg_x_body(x_hbm, src_hbm, dst_hbm, zr_hbm, agg_hbm,
                   s0, s1, s2, s3, d0, d1, d2, d3, r0, r1, r2, r3, acc,
                   m0, m1, m2, m3):
    core = lax.axis_index("c")
    sid = lax.axis_index("s")
    tid = core * NS + sid
    base = sid * RPT
    ebase = tid * EPT
    issue, drain_scatter = _ring_agg(
        x_hbm, src_hbm, dst_hbm, ebase,
        (s0, s1, s2, s3), (d0, d1, d2, d3), (r0, r1, r2, r3),
        (m0, m1, m2, m3), acc)

    pltpu.sync_copy(zr_hbm, acc.at[pl.ds(base, RPT)])
    for b in range(RING):
        issue(b, b)
    plsc.subcore_barrier()

    @pl.loop(0, NMAIN - RING, step=RING)
    def _(v):
        for b in range(RING):
            drain_scatter(b)
            issue(b, v + b + RING)

    for b in range(RING):
        drain_scatter(b)
    for j in range(NMAIN, NBATCH):
        issue(0, j)
        drain_scatter(0)

    plsc.subcore_barrier()
    pltpu.sync_copy(acc.at[pl.ds(base, RPT)],
                    agg_hbm.at[core].at[pl.ds(base, RPT)])


def _ring_scratch():
    return ([pltpu.VMEM((EB,), jnp.int32)] * (2 * RING)
            + [pltpu.VMEM((EB, IN_CH), _f32)] * RING)


def _sc_agg_x(x, src_r, dst_r, zrows):
    k = pl.kernel(
        _sc_agg_x_body,
        out_type=jax.ShapeDtypeStruct((NC, NPAD, IN_CH), _f32),
        mesh=plsc.VectorSubcoreMesh(**_sc_mesh),
        scratch_types=_ring_scratch()
        + [pltpu.VMEM_SHARED((NPAD, IN_CH), _f32)]
        + [pltpu.SemaphoreType.DMA] * RING,
    )
    return k(x, src_r, dst_r, zrows)


def _sc_agg_h_body(h0_hbm, h1_hbm, h2_hbm, h3_hbm, src_hbm, dst_hbm, zr_hbm,
                   agg_hbm, s0, s1, s2, s3, d0, d1, d2, d3, r0, r1, r2, r3,
                   acc, m0, m1, m2, m3):
    core = lax.axis_index("c")
    sid = lax.axis_index("s")
    tid = core * NS + sid
    base = sid * RPT
    ebase = tid * EPT

    for c, hc in enumerate((h0_hbm, h1_hbm, h2_hbm, h3_hbm)):
        issue, drain_scatter = _ring_agg(
            hc, src_hbm, dst_hbm, ebase,
            (s0, s1, s2, s3), (d0, d1, d2, d3), (r0, r1, r2, r3),
            (m0, m1, m2, m3), acc)

        pltpu.sync_copy(zr_hbm, acc.at[pl.ds(base, RPT)])
        for b in range(RING):
            issue(b, b)
        plsc.subcore_barrier()

        @pl.loop(0, NMAIN - RING, step=RING)
        def _(v):
            for b in range(RING):
                drain_scatter(b)
                issue(b, v + b + RING)

        for b in range(RING):
            drain_scatter(b)
        for j in range(NMAIN, NBATCH):
            issue(0, j)
            drain_scatter(0)

        plsc.subcore_barrier()
        pltpu.sync_copy(acc.at[pl.ds(base, RPT)],
                        agg_hbm.at[c].at[core].at[pl.ds(base, RPT)])


def _sc_agg_h(h0, h1, h2, h3, src_r, dst_r, zrows):
    k = pl.kernel(
        _sc_agg_h_body,
        out_type=jax.ShapeDtypeStruct((4, NC, NPAD, IN_CH), _f32),
        mesh=plsc.VectorSubcoreMesh(**_sc_mesh),
        scratch_types=_ring_scratch()
        + [pltpu.VMEM_SHARED((NPAD, IN_CH), _f32)]
        + [pltpu.SemaphoreType.DMA] * RING,
    )
    return k(h0, h1, h2, h3, src_r, dst_r, zrows)


NB = 512


def _tc_a_body(x_ref, agg_ref, deg_ref, wflat_ref, br_ref, g_ref, b_ref,
               sW_ref, sb_ref, h0_ref, h1_ref, h2_ref, h3_ref):
    ha = agg_ref[0] + agg_ref[1]
    xb = x_ref[...]
    degf = deg_ref[0] + deg_ref[1]
    d = jnp.clip(degf[:, 0:1].astype(jnp.int32), 0, MAX_DEG)
    oh = (d == lax.broadcasted_iota(jnp.int32, (1, MAX_DEG + 1), 1))
    oh = oh.astype(_f32)
    hx = jnp.concatenate([ha, xb], axis=1)
    z = jnp.concatenate([oh[:, dd:dd + 1] * hx for dd in range(MAX_DEG + 1)],
                        axis=1)
    mf = jnp.dot(z, wflat_ref[...], preferred_element_type=_f32)
    mf = mf + jnp.dot(oh, br_ref[...], preferred_element_type=_f32)
    y = jnp.maximum(mf * g_ref[...] + b_ref[...], 0.0)
    h = jnp.dot(y, sW_ref[...], preferred_element_type=_f32) + sb_ref[...]
    h = jnp.maximum(h, 0.0)
    h0_ref[...] = h[:, 0:128]
    h1_ref[...] = h[:, 128:256]
    h2_ref[...] = h[:, 256:384]
    h3_ref[...] = h[:, 384:512]


def _tc_a(x_pad, aggx, dega, wflat, br, gscale, bshift, s1_W, s1_bias):
    grid = (NPAD // NB,)
    const = lambda i: (0, 0)
    out_sd = jax.ShapeDtypeStruct((NPAD, IN_CH), _f32)
    return pl.pallas_call(
        _tc_a_body,
        grid=grid,
        in_specs=[
            pl.BlockSpec((NB, IN_CH), lambda i: (i, 0)),
            pl.BlockSpec((NC, NB, IN_CH), lambda i: (0, i, 0)),
            pl.BlockSpec((NC, NB, IN_CH), lambda i: (0, i, 0)),
            pl.BlockSpec(((MAX_DEG + 1) * 256, C4), const),
            pl.BlockSpec((MAX_DEG + 1, C4), const),
            pl.BlockSpec((1, C4), const),
            pl.BlockSpec((1, C4), const),
            pl.BlockSpec((C4, C4), const),
            pl.BlockSpec((1, C4), const),
        ],
        out_specs=[pl.BlockSpec((NB, IN_CH), lambda i: (i, 0))] * 4,
        out_shape=[out_sd] * 4,
    )(x_pad, aggx, dega, wflat, br, gscale, bshift, s1_W, s1_bias)


def _tc_b_body(h0, h1, h2, h3, aggh_ref,
               c2W1, c2b1, c2g, c2be, c2W2, c2b2,
               c3W1, c3b1, c3g, c3be, c3W2, c3b2,
               l1a, l1b_, l1bias, l2W, l2b, oW, ob, out_ref):
    hin = jnp.concatenate(
        [h0[...] + aggh_ref[0, 0] + aggh_ref[0, 1],
         h1[...] + aggh_ref[1, 0] + aggh_ref[1, 1],
         h2[...] + aggh_ref[2, 0] + aggh_ref[2, 1],
         h3[...] + aggh_ref[3, 0] + aggh_ref[3, 1]], axis=1)

    def gin(W1, b1, g, be, W2, b2):
        t = jnp.dot(hin, W1[...], preferred_element_type=_f32) + b1[...]
        t = jnp.maximum(t * g[...] + be[...], 0.0)
        t = jnp.dot(t, W2[...], preferred_element_type=_f32) + b2[...]
        return jnp.maximum(t, 0.0)

    x1 = gin(c2W1, c2b1, c2g, c2be, c2W2, c2b2)
    x2 = gin(c3W1, c3b1, c3g, c3be, c3W2, c3b2)
    y = (jnp.dot(x1, l1a[...], preferred_element_type=_f32)
         + jnp.dot(x2, l1b_[...], preferred_element_type=_f32) + l1bias[...])
    y = jnp.dot(y, l2W[...], preferred_element_type=_f32) + l2b[...]
    y = jnp.dot(y, oW[...], preferred_element_type=_f32) + ob[...]
    out_ref[...] = jax.nn.sigmoid(y)


def _tc_b(h_chunks, aggh, weights):
    grid = (NPAD // NB,)
    const = lambda i: (0, 0)
    wspecs = []
    for w in weights:
        if w.ndim == 2:
            wspecs.append(pl.BlockSpec(w.shape, const))
        else:
            raise AssertionError
    return pl.pallas_call(
        _tc_b_body,
        grid=grid,
        in_specs=[pl.BlockSpec((NB, IN_CH), lambda i: (i, 0))] * 4
        + [pl.BlockSpec((4, NC, NB, IN_CH), lambda i: (0, 0, i, 0))]
        + wspecs,
        out_specs=pl.BlockSpec((NB, IN_CH), lambda i: (i, 0)),
        out_shape=jax.ShapeDtypeStruct((NPAD, IN_CH), _f32),
    )(*h_chunks, aggh, *weights)


def kernel(x, c1_Wn, c1_Wr, c1_br, s1_g, s1_b, s1_W, s1_bias,
           c2_W1, c2_b1, c2_g, c2_beta, c2_W2, c2_b2,
           c3_W1, c3_b1, c3_g, c3_beta, c3_W2, c3_b2,
           l1_W, l1_b, l2_W, l2_b, o_W, o_b, edge_index, batch):
    del batch
    inv = 1.0 / jnp.sqrt(jnp.float32(1.0 + 1e-5))
    src_r = edge_index[0]
    dst_r = edge_index[1]
    x_pad = jnp.pad(x, ((0, NPAD - N_NODES), (0, 0)))
    zrows = jnp.zeros((RPT, IN_CH), _f32)
    ones_rows = jnp.ones((EBD, IN_CH), _f32)

    dega = _sc_deg(dst_r, zrows, ones_rows)
    aggx = _sc_agg_x(x_pad, src_r, dst_r, zrows)

    wflat = jnp.concatenate([c1_Wn, c1_Wr], axis=1).reshape(
        (MAX_DEG + 1) * 256, C4)
    h0, h1, h2, h3 = _tc_a(
        x_pad, aggx, dega, wflat, c1_br,
        (s1_g * inv).reshape(1, C4), s1_b.reshape(1, C4),
        s1_W, s1_bias.reshape(1, C4))

    aggh = _sc_agg_h(h0, h1, h2, h3, src_r, dst_r, zrows)

    weights = (
        c2_W1, c2_b1.reshape(1, C8), (c2_g * inv).reshape(1, C8),
        c2_beta.reshape(1, C8), c2_W2, c2_b2.reshape(1, C8),
        c3_W1, c3_b1.reshape(1, C8), (c3_g * inv).reshape(1, C8),
        c3_beta.reshape(1, C8), c3_W2, c3_b2.reshape(1, C8),
        l1_W[:C8], l1_W[C8:], l1_b.reshape(1, C8),
        l2_W, l2_b.reshape(1, C4), o_W, o_b.reshape(1, IN_CH),
    )
    out = _tc_b((h0, h1, h2, h3), aggh, weights)
    return out[:N_NODES]


# speedup vs baseline: 4.7831x; 1.0001x over previous
import jax
import jax.numpy as jnp
from jax import lax
from jax.experimental import pallas as pl
from jax.experimental.pallas import tpu as pltpu
from jax.experimental.pallas import tpu_sc as plsc

N_NODES = 10000
N_EDGES = 320000
IN_CH = 128
C4 = 512
C8 = 1024
MAX_DEG = 10
NPAD = 10240

NC = 2
NS = 16
NT = NC * NS
EPT = N_EDGES // NT
EB = 80
NBATCH = EPT // EB
RING = 4
NMAIN = (NBATCH // RING) * RING
DW = 128
EBD = 200
NBD = EPT // EBD
RPT = NPAD // NS

_f32 = jnp.float32
_sc_mesh = dict(core_axis_name="c", subcore_axis_name="s")


def _sc_deg_body(dst_hbm, zr_hbm, ones_hbm, deg_hbm, ones_v, dst_v, dacc, sem):
    core = lax.axis_index("c")
    sid = lax.axis_index("s")
    tid = core * NS + sid
    base = sid * RPT

    pltpu.sync_copy(ones_hbm, ones_v)
    pltpu.sync_copy(zr_hbm, dacc.at[pl.ds(base, RPT)])
    plsc.subcore_barrier()
    ebase = tid * EPT

    @pl.loop(0, NBD)
    def _(i):
        off = ebase + i * EBD
        pltpu.sync_copy(dst_hbm.at[pl.ds(off, EBD)], dst_v)
        pltpu.sync_copy(ones_v, dacc.at[dst_v], add=True)

    plsc.subcore_barrier()
    pltpu.sync_copy(dacc.at[pl.ds(base, RPT)],
                    deg_hbm.at[core].at[pl.ds(base, RPT)])


def _sc_deg(dst_r, zdeg, ones_rows):
    k = pl.kernel(
        _sc_deg_body,
        out_type=jax.ShapeDtypeStruct((NC, NPAD, DW), _f32),
        mesh=plsc.VectorSubcoreMesh(**_sc_mesh),
        scratch_types=[
            pltpu.VMEM((EBD, DW), _f32),
            pltpu.VMEM((EBD,), jnp.int32),
            pltpu.VMEM_SHARED((NPAD, DW), _f32),
            pltpu.SemaphoreType.DMA,
        ],
    )
    return k(dst_r, zdeg, ones_rows)


def _ring_agg(table_hbm, src_hbm, dst_hbm, ebase, srcs, dsts, rows, sems,
              acc):
    def issue(b, i):
        off = ebase + i * EB
        pltpu.sync_copy(src_hbm.at[pl.ds(off, EB)], srcs[b])
        pltpu.sync_copy(dst_hbm.at[pl.ds(off, EB)], dsts[b])
        pltpu.async_copy(table_hbm.at[srcs[b]], rows[b], sems[b])

    def drain_scatter(b):
        pltpu.make_async_copy(table_hbm.at[srcs[b]], rows[b], sems[b]).wait()
        pltpu.sync_copy(rows[b], acc.at[dsts[b]], add=True)

    return issue, drain_scatter


def _sc_agg_x_body(x_hbm, src_hbm, dst_hbm, zr_hbm, agg_hbm,
                   s0, s1, s2, s3, d0, d1, d2, d3, r0, r1, r2, r3, acc,
                   m0, m1, m2, m3):
    core = lax.axis_index("c")
    sid = lax.axis_index("s")
    tid = core * NS + sid
    base = sid * RPT
    ebase = tid * EPT
    issue, drain_scatter = _ring_agg(
        x_hbm, src_hbm, dst_hbm, ebase,
        (s0, s1, s2, s3), (d0, d1, d2, d3), (r0, r1, r2, r3),
        (m0, m1, m2, m3), acc)

    pltpu.sync_copy(zr_hbm, acc.at[pl.ds(base, RPT)])
    for b in range(RING):
        issue(b, b)
    plsc.subcore_barrier()

    @pl.loop(0, NMAIN - RING, step=RING)
    def _(v):
        for b in range(RING):
            drain_scatter(b)
            issue(b, v + b + RING)

    for b in range(RING):
        drain_scatter(b)
    for j in range(NMAIN, NBATCH):
        issue(0, j)
        drain_scatter(0)

    plsc.subcore_barrier()
    pltpu.sync_copy(acc.at[pl.ds(base, RPT)],
                    agg_hbm.at[core].at[pl.ds(base, RPT)])


def _ring_scratch():
    return ([pltpu.VMEM((EB,), jnp.int32)] * (2 * RING)
            + [pltpu.VMEM((EB, IN_CH), _f32)] * RING)


def _sc_agg_x(x, src_r, dst_r, zrows):
    k = pl.kernel(
        _sc_agg_x_body,
        out_type=jax.ShapeDtypeStruct((NC, NPAD, IN_CH), _f32),
        mesh=plsc.VectorSubcoreMesh(**_sc_mesh),
        scratch_types=_ring_scratch()
        + [pltpu.VMEM_SHARED((NPAD, IN_CH), _f32)]
        + [pltpu.SemaphoreType.DMA] * RING,
    )
    return k(x, src_r, dst_r, zrows)


def _sc_agg_h_body(h0_hbm, h1_hbm, h2_hbm, h3_hbm, src_hbm, dst_hbm, zr_hbm,
                   agg_hbm, s0, s1, s2, s3, d0, d1, d2, d3, r0, r1, r2, r3,
                   acc, m0, m1, m2, m3):
    core = lax.axis_index("c")
    sid = lax.axis_index("s")
    tid = core * NS + sid
    base = sid * RPT
    ebase = tid * EPT
    srcs = (s0, s1, s2, s3)
    dsts = (d0, d1, d2, d3)
    rows = (r0, r1, r2, r3)
    sems = (m0, m1, m2, m3)

    for c, hc in enumerate((h0_hbm, h1_hbm, h2_hbm, h3_hbm)):
        issue, drain_scatter = _ring_agg(
            hc, src_hbm, dst_hbm, ebase, srcs, dsts, rows, sems, acc)

        pltpu.sync_copy(zr_hbm, acc.at[pl.ds(base, RPT)])
        for b in range(RING):
            issue(b, b)
        plsc.subcore_barrier()

        @pl.loop(0, NMAIN - RING, step=RING)
        def _(v):
            for b in range(RING):
                drain_scatter(b)
                issue(b, v + b + RING)

        for b in range(RING):
            drain_scatter(b)
        for j in range(NMAIN, NBATCH):
            issue(0, j)
            drain_scatter(0)

        plsc.subcore_barrier()
        pltpu.sync_copy(acc.at[pl.ds(base, RPT)],
                        agg_hbm.at[c].at[core].at[pl.ds(base, RPT)])


def _sc_agg_h(h0, h1, h2, h3, src_r, dst_r, zrows):
    k = pl.kernel(
        _sc_agg_h_body,
        out_type=jax.ShapeDtypeStruct((4, NC, NPAD, IN_CH), _f32),
        mesh=plsc.VectorSubcoreMesh(**_sc_mesh),
        scratch_types=_ring_scratch()
        + [pltpu.VMEM_SHARED((NPAD, IN_CH), _f32)]
        + [pltpu.SemaphoreType.DMA] * RING,
    )
    return k(h0, h1, h2, h3, src_r, dst_r, zrows)


NB = 512


def _tc_a_body(x_ref, agg_ref, deg_ref, wflat_ref, br_ref, g_ref, b_ref,
               sW_ref, sb_ref, h0_ref, h1_ref, h2_ref, h3_ref):
    ha = agg_ref[0] + agg_ref[1]
    xb = x_ref[...]
    degf = deg_ref[0] + deg_ref[1]
    d = jnp.clip(degf[:, 0:1].astype(jnp.int32), 0, MAX_DEG)
    oh = (d == lax.broadcasted_iota(jnp.int32, (1, MAX_DEG + 1), 1))
    oh = oh.astype(_f32)
    hx = jnp.concatenate([ha, xb], axis=1)
    z = jnp.concatenate([oh[:, dd:dd + 1] * hx for dd in range(MAX_DEG + 1)],
                        axis=1)
    mf = jnp.dot(z, wflat_ref[...], preferred_element_type=_f32)
    mf = mf + jnp.dot(oh, br_ref[...], preferred_element_type=_f32)
    y = jnp.maximum(mf * g_ref[...] + b_ref[...], 0.0)
    h = jnp.dot(y, sW_ref[...], preferred_element_type=_f32) + sb_ref[...]
    h = jnp.maximum(h, 0.0)

    for c, fref in enumerate((h0_ref, h1_ref, h2_ref, h3_ref)):
        fref[...] = h[:, c * 128:(c + 1) * 128]


def _tc_a(x_pad, aggx, dega, wflat, br, gscale, bshift, s1_W, s1_bias):
    grid = (NPAD // NB,)
    const = lambda i: (0, 0)
    out_sd = jax.ShapeDtypeStruct((NPAD, IN_CH), _f32)
    return pl.pallas_call(
        _tc_a_body,
        grid=grid,
        in_specs=[
            pl.BlockSpec((NB, IN_CH), lambda i: (i, 0)),
            pl.BlockSpec((NC, NB, IN_CH), lambda i: (0, i, 0)),
            pl.BlockSpec((NC, NB, DW), lambda i: (0, i, 0)),
            pl.BlockSpec(((MAX_DEG + 1) * 256, C4), const),
            pl.BlockSpec((MAX_DEG + 1, C4), const),
            pl.BlockSpec((1, C4), const),
            pl.BlockSpec((1, C4), const),
            pl.BlockSpec((C4, C4), const),
            pl.BlockSpec((1, C4), const),
        ],
        out_specs=[pl.BlockSpec((NB, IN_CH), lambda i: (i, 0))] * 4,
        out_shape=[out_sd] * 4,
    )(x_pad, aggx, dega, wflat, br, gscale, bshift, s1_W, s1_bias)


def _tc_b_body(h0, h1, h2, h3, aggh_ref,
               c2W1, c2b1, c2g, c2be, c2W2, c2b2,
               c3W1, c3b1, c3g, c3be, c3W2, c3b2,
               l1a, l1b_, l1bias, l2W, l2b, oW, ob, out_ref):
    hin = jnp.concatenate(
        [h0[...] + aggh_ref[0, 0] + aggh_ref[0, 1],
         h1[...] + aggh_ref[1, 0] + aggh_ref[1, 1],
         h2[...] + aggh_ref[2, 0] + aggh_ref[2, 1],
         h3[...] + aggh_ref[3, 0] + aggh_ref[3, 1]], axis=1)

    def gin(W1, b1, g, be, W2, b2):
        t = jnp.dot(hin, W1[...], preferred_element_type=_f32) + b1[...]
        t = jnp.maximum(t * g[...] + be[...], 0.0)
        t = jnp.dot(t, W2[...], preferred_element_type=_f32) + b2[...]
        return jnp.maximum(t, 0.0)

    x1 = gin(c2W1, c2b1, c2g, c2be, c2W2, c2b2)
    x2 = gin(c3W1, c3b1, c3g, c3be, c3W2, c3b2)
    y = (jnp.dot(x1, l1a[...], preferred_element_type=_f32)
         + jnp.dot(x2, l1b_[...], preferred_element_type=_f32) + l1bias[...])
    y = jnp.dot(y, l2W[...], preferred_element_type=_f32) + l2b[...]
    y = jnp.dot(y, oW[...], preferred_element_type=_f32) + ob[...]
    out_ref[...] = jax.nn.sigmoid(y)


def _tc_b(h_chunks, aggh, weights):
    grid = (NPAD // NB,)
    const = lambda i: (0, 0)
    wspecs = []
    for w in weights:
        if w.ndim == 2:
            wspecs.append(pl.BlockSpec(w.shape, const))
        else:
            raise AssertionError
    return pl.pallas_call(
        _tc_b_body,
        grid=grid,
        in_specs=[pl.BlockSpec((NB, IN_CH), lambda i: (i, 0))] * 4
        + [pl.BlockSpec((4, NC, NB, IN_CH), lambda i: (0, 0, i, 0))]
        + wspecs,
        out_specs=pl.BlockSpec((NB, IN_CH), lambda i: (i, 0)),
        out_shape=jax.ShapeDtypeStruct((NPAD, IN_CH), _f32),
    )(*h_chunks, aggh, *weights)


def kernel(x, c1_Wn, c1_Wr, c1_br, s1_g, s1_b, s1_W, s1_bias,
           c2_W1, c2_b1, c2_g, c2_beta, c2_W2, c2_b2,
           c3_W1, c3_b1, c3_g, c3_beta, c3_W2, c3_b2,
           l1_W, l1_b, l2_W, l2_b, o_W, o_b, edge_index, batch):
    del batch
    inv = 1.0 / jnp.sqrt(jnp.float32(1.0 + 1e-5))
    src_r = edge_index[0]
    dst_r = edge_index[1]
    x_pad = jnp.pad(x, ((0, NPAD - N_NODES), (0, 0)))
    zrows = jnp.zeros((RPT, IN_CH), _f32)
    ones_rows = jnp.ones((EBD, DW), _f32)
    zdeg = jnp.zeros((RPT, DW), _f32)

    dega = _sc_deg(dst_r, zdeg, ones_rows)
    aggx = _sc_agg_x(x_pad, src_r, dst_r, zrows)

    wflat = jnp.concatenate([c1_Wn, c1_Wr], axis=1).reshape(
        (MAX_DEG + 1) * 256, C4)
    h0, h1, h2, h3 = _tc_a(
        x_pad, aggx, dega, wflat, c1_br,
        (s1_g * inv).reshape(1, C4), s1_b.reshape(1, C4),
        s1_W, s1_bias.reshape(1, C4))

    aggh = _sc_agg_h(h0, h1, h2, h3, src_r, dst_r, zrows)

    weights = (
        c2_W1, c2_b1.reshape(1, C8), (c2_g * inv).reshape(1, C8),
        c2_beta.reshape(1, C8), c2_W2, c2_b2.reshape(1, C8),
        c3_W1, c3_b1.reshape(1, C8), (c3_g * inv).reshape(1, C8),
        c3_beta.reshape(1, C8), c3_W2, c3_b2.reshape(1, C8),
        l1_W[:C8], l1_W[C8:], l1_b.reshape(1, C8),
        l2_W, l2_b.reshape(1, C4), o_W, o_b.reshape(1, IN_CH),
    )
    out = _tc_b((h0, h1, h2, h3), aggh, weights)
    return out[:N_NODES]
